# baseline (speedup 1.0000x reference)
import functools

import jax
import jax.numpy as jnp
from jax import lax
from jax.experimental import pallas as pl
from jax.experimental.pallas import tpu as pltpu
from jax.experimental.pallas import tpu_sc as plsc

IDX_ROW = 128
ROWS_PER_GROUP = 4


@functools.lru_cache(maxsize=None)
def _make_gather(num_idx_rows: int, d: int):
    info = plsc.get_sparse_core_info()
    nc, ns = info.num_cores, info.num_subcores
    nw = nc * ns
    rows_per_w = num_idx_rows // nw
    num_groups = rows_per_w // ROWS_PER_GROUP
    assert num_groups % 2 == 0 and num_groups >= 4
    group_rows = ROWS_PER_GROUP * IDX_ROW

    mesh = plsc.VectorSubcoreMesh(core_axis_name="c", subcore_axis_name="s")

    @functools.partial(
        pl.kernel,
        mesh=mesh,
        out_type=jax.ShapeDtypeStruct((num_idx_rows * IDX_ROW, d), jnp.float32),
        scratch_types=[
            pltpu.VMEM((rows_per_w, IDX_ROW), jnp.int32),
            pltpu.VMEM((group_rows, d), jnp.float32),
            pltpu.VMEM((group_rows, d), jnp.float32),
            pltpu.SemaphoreType.DMA,
            pltpu.SemaphoreType.DMA,
            pltpu.SemaphoreType.DMA,
            pltpu.SemaphoreType.DMA,
        ],
        compiler_params=pltpu.CompilerParams(
            use_tc_tiling_on_sc=False, skip_device_barrier=True
        ),
    )
    def gather_kernel(table_hbm, idx_hbm, out_hbm, idx_v, buf0, buf1,
                      gsem0, gsem1, wsem0, wsem1):
        wid = lax.axis_index("s") * nc + lax.axis_index("c")
        base_row = wid * rows_per_w
        bufs = (buf0, buf1)
        gsems = (gsem0, gsem1)
        wsems = (wsem0, wsem1)

        pltpu.sync_copy(idx_hbm.at[pl.ds(base_row, rows_per_w)], idx_v)

        def fire_gathers(g, b):
            j0 = g * ROWS_PER_GROUP
            for q in range(ROWS_PER_GROUP):
                pltpu.async_copy(
                    table_hbm.at[idx_v.at[j0 + q]],
                    bufs[b].at[pl.ds(q * IDX_ROW, IDX_ROW)],
                    gsems[b],
                )

        def wait_gathers(b):
            pltpu.make_async_copy(
                out_hbm.at[pl.ds(0, group_rows)], bufs[b], gsems[b]
            ).wait()

        def fire_write(g, b):
            pltpu.async_copy(
                bufs[b],
                out_hbm.at[pl.ds((base_row + g * ROWS_PER_GROUP) * IDX_ROW,
                                 group_rows)],
                wsems[b],
            )

        def wait_write(b):
            pltpu.make_async_copy(
                bufs[b], out_hbm.at[pl.ds(0, group_rows)], wsems[b]
            ).wait()

        fire_gathers(0, 0)
        fire_gathers(1, 1)
        wait_gathers(0)
        fire_write(0, 0)

        def round_body(r, carry):
            wait_write(0)
            fire_gathers(2 * r, 0)
            wait_gathers(1)
            fire_write(2 * r - 1, 1)
            wait_write(1)
            fire_gathers(2 * r + 1, 1)
            wait_gathers(0)
            fire_write(2 * r, 0)
            return carry

        lax.fori_loop(1, num_groups // 2, round_body, 0)

        wait_gathers(1)
        fire_write(num_groups - 1, 1)
        wait_write(0)
        wait_write(1)

    return gather_kernel


def kernel(x, weight):
    b, h = x.shape
    n = b * h
    d = weight.shape[1]
    idx2d = x.reshape(n // IDX_ROW, IDX_ROW).astype(jnp.int32)
    out = _make_gather(n // IDX_ROW, d)(weight, idx2d)
    return out.reshape(b, h, d)


# speedup vs baseline: 1.2232x; 1.2232x over previous
import functools

import jax
import jax.numpy as jnp
from jax import lax
from jax.experimental import pallas as pl
from jax.experimental.pallas import tpu as pltpu
from jax.experimental.pallas import tpu_sc as plsc

IDX_ROW = 128
ROWS_PER_GROUP = 2
PAD_D = 128


@functools.lru_cache(maxsize=None)
def _make_gather(num_idx_rows: int):
    info = plsc.get_sparse_core_info()
    nc, ns = info.num_cores, info.num_subcores
    nw = nc * ns
    rows_per_w = num_idx_rows // nw
    num_groups = rows_per_w // ROWS_PER_GROUP
    assert num_groups % 2 == 0 and num_groups >= 4
    group_rows = ROWS_PER_GROUP * IDX_ROW

    mesh = plsc.VectorSubcoreMesh(core_axis_name="c", subcore_axis_name="s")

    @functools.partial(
        pl.kernel,
        mesh=mesh,
        out_type=jax.ShapeDtypeStruct((num_idx_rows * IDX_ROW, PAD_D),
                                      jnp.float32),
        scratch_types=[
            pltpu.VMEM((rows_per_w, IDX_ROW), jnp.int32),
            pltpu.VMEM((group_rows, PAD_D), jnp.float32),
            pltpu.VMEM((group_rows, PAD_D), jnp.float32),
            pltpu.SemaphoreType.DMA,
            pltpu.SemaphoreType.DMA,
            pltpu.SemaphoreType.DMA,
            pltpu.SemaphoreType.DMA,
        ],
        compiler_params=pltpu.CompilerParams(use_tc_tiling_on_sc=False),
    )
    def gather_kernel(table_hbm, idx_hbm, out_hbm, idx_v, buf0, buf1,
                      gsem0, gsem1, wsem0, wsem1):
        wid = lax.axis_index("s") * nc + lax.axis_index("c")
        base_row = wid * rows_per_w
        bufs = (buf0, buf1)
        gsems = (gsem0, gsem1)
        wsems = (wsem0, wsem1)

        pltpu.sync_copy(idx_hbm.at[pl.ds(base_row, rows_per_w)], idx_v)

        def fire_gathers(g, b):
            j0 = g * ROWS_PER_GROUP
            for q in range(ROWS_PER_GROUP):
                pltpu.async_copy(
                    table_hbm.at[idx_v.at[j0 + q]],
                    bufs[b].at[pl.ds(q * IDX_ROW, IDX_ROW)],
                    gsems[b],
                )

        def wait_gathers(b):
            pltpu.make_async_copy(
                out_hbm.at[pl.ds(0, group_rows)], bufs[b], gsems[b]
            ).wait()

        def fire_write(g, b):
            pltpu.async_copy(
                bufs[b],
                out_hbm.at[pl.ds((base_row + g * ROWS_PER_GROUP) * IDX_ROW,
                                 group_rows)],
                wsems[b],
            )

        def wait_write(b):
            pltpu.make_async_copy(
                bufs[b], out_hbm.at[pl.ds(0, group_rows)], wsems[b]
            ).wait()

        fire_gathers(0, 0)
        fire_gathers(1, 1)
        wait_gathers(0)
        fire_write(0, 0)

        def round_body(r, carry):
            wait_write(0)
            fire_gathers(2 * r, 0)
            wait_gathers(1)
            fire_write(2 * r - 1, 1)
            wait_write(1)
            fire_gathers(2 * r + 1, 1)
            wait_gathers(0)
            fire_write(2 * r, 0)
            return carry

        lax.fori_loop(1, num_groups // 2, round_body, 0)

        wait_gathers(1)
        fire_write(num_groups - 1, 1)
        wait_write(0)
        wait_write(1)

    return gather_kernel


def kernel(x, weight):
    b, h = x.shape
    n = b * h
    d = weight.shape[1]
    wp = jnp.pad(weight, ((0, 0), (0, PAD_D - d)))
    idx2d = x.reshape(n // IDX_ROW, IDX_ROW).astype(jnp.int32)
    out2 = _make_gather(n // IDX_ROW)(wp, idx2d)
    return out2[:, :d].reshape(b, h, d)


# speedup vs baseline: 1.4296x; 1.1688x over previous
import functools

import jax
import jax.numpy as jnp
from jax import lax
from jax.experimental import pallas as pl
from jax.experimental.pallas import tpu as pltpu
from jax.experimental.pallas import tpu_sc as plsc

IDX_ROW = 128
ROWS_PER_GROUP = 4
PAD_D = 128


@functools.lru_cache(maxsize=None)
def _make_gather(num_idx_rows: int, d: int):
    info = plsc.get_sparse_core_info()
    nc, ns = info.num_cores, info.num_subcores
    nw = nc * ns
    rows_per_w = num_idx_rows // nw
    num_groups = rows_per_w // ROWS_PER_GROUP
    assert num_groups % 2 == 0 and num_groups >= 4
    group_rows = ROWS_PER_GROUP * IDX_ROW

    mesh = plsc.VectorSubcoreMesh(core_axis_name="c", subcore_axis_name="s")

    @functools.partial(
        pl.kernel,
        mesh=mesh,
        out_type=jax.ShapeDtypeStruct((num_idx_rows * IDX_ROW, PAD_D),
                                      jnp.float32),
        scratch_types=[
            pltpu.VMEM((rows_per_w, IDX_ROW), jnp.int32),
            pltpu.VMEM((group_rows, d), jnp.float32),
            pltpu.VMEM((group_rows, d), jnp.float32),
            pltpu.SemaphoreType.DMA,
            pltpu.SemaphoreType.DMA,
            pltpu.SemaphoreType.DMA,
            pltpu.SemaphoreType.DMA,
        ],
        compiler_params=pltpu.CompilerParams(use_tc_tiling_on_sc=False),
    )
    def gather_kernel(table_hbm, idx_hbm, out_hbm, idx_v, buf0, buf1,
                      gsem0, gsem1, wsem0, wsem1):
        wid = lax.axis_index("s") * nc + lax.axis_index("c")
        base_row = wid * rows_per_w
        bufs = (buf0, buf1)
        gsems = (gsem0, gsem1)
        wsems = (wsem0, wsem1)

        pltpu.sync_copy(idx_hbm.at[pl.ds(base_row, rows_per_w)], idx_v)

        def fire_gathers(g, b):
            j0 = g * ROWS_PER_GROUP
            for q in range(ROWS_PER_GROUP):
                pltpu.async_copy(
                    table_hbm.at[idx_v.at[j0 + q]],
                    bufs[b].at[pl.ds(q * IDX_ROW, IDX_ROW)],
                    gsems[b],
                )

        def wait_gathers(b):
            pltpu.make_async_copy(
                out_hbm.at[pl.ds(0, group_rows), pl.ds(0, d)],
                bufs[b], gsems[b]
            ).wait()

        def fire_write(g, b):
            pltpu.async_copy(
                bufs[b],
                out_hbm.at[pl.ds((base_row + g * ROWS_PER_GROUP) * IDX_ROW,
                                 group_rows), pl.ds(0, d)],
                wsems[b],
            )

        def wait_write(b):
            pltpu.make_async_copy(
                bufs[b], out_hbm.at[pl.ds(0, group_rows), pl.ds(0, d)],
                wsems[b]
            ).wait()

        fire_gathers(0, 0)
        fire_gathers(1, 1)
        wait_gathers(0)
        fire_write(0, 0)

        def round_body(r, carry):
            wait_write(0)
            fire_gathers(2 * r, 0)
            wait_gathers(1)
            fire_write(2 * r - 1, 1)
            wait_write(1)
            fire_gathers(2 * r + 1, 1)
            wait_gathers(0)
            fire_write(2 * r, 0)
            return carry

        lax.fori_loop(1, num_groups // 2, round_body, 0)

        wait_gathers(1)
        fire_write(num_groups - 1, 1)
        wait_write(0)
        wait_write(1)

    return gather_kernel


def kernel(x, weight):
    b, h = x.shape
    n = b * h
    d = weight.shape[1]
    wp = jnp.pad(weight, ((0, 0), (0, PAD_D - d))).reshape(-1, d)
    idx2d = (x.reshape(n // IDX_ROW, IDX_ROW) * 2).astype(jnp.int32)
    out2 = _make_gather(n // IDX_ROW, d)(wp, idx2d)
    return out2[:, :d].reshape(b, h, d)
